# baseline (device time: 30683 ns/iter reference)
import jax
import jax.numpy as jnp
from jax import lax
from jax.experimental import pallas as pl
from jax.experimental.pallas import tpu as pltpu

N_DEV = 4
BLK = 64


def kernel(x, Wq, K_ext, V_ext, Wo):
    B, sq, dm = x.shape
    dq = Wq.shape[1]
    skv, hq, dh = K_ext.shape[1:]
    do = Wo.shape[1]
    n_blk = sq // BLK
    h_per = dq // dh
    bf = jnp.bfloat16

    wq_bf = Wq.astype(bf)
    wo_bf = Wo.astype(bf)

    def body(x_ref, wq_ref, k_ref, v_ref, wo_ref, out_ref,
             wq_full, wo_full, k_bf, v_bf, send_sems, recv_sems):
        my = lax.axis_index("i")

        barrier = pltpu.get_barrier_semaphore()
        for p in range(1, N_DEV):
            pl.semaphore_signal(
                barrier, inc=1,
                device_id=((my + p) % N_DEV,),
                device_id_type=pl.DeviceIdType.MESH,
            )
        pl.semaphore_wait(barrier, N_DEV - 1)

        sends = []
        for p in range(1, N_DEV):
            tgt = (my + p) % N_DEV
            for idx, (src, dst) in enumerate((
                (wq_ref, wq_full.at[:, pl.ds(my * dq, dq)]),
                (wo_ref, wo_full.at[pl.ds(my * dq, dq), :]),
            )):
                rdma = pltpu.make_async_remote_copy(
                    src_ref=src, dst_ref=dst,
                    send_sem=send_sems.at[2 * (p - 1) + idx],
                    recv_sem=recv_sems.at[2 * (p - 1) + idx],
                    device_id=(tgt,), device_id_type=pl.DeviceIdType.MESH,
                )
                rdma.start()
                sends.append(rdma)

        k_bf[...] = k_ref[...].reshape(B * skv, hq * dh).astype(bf)
        v_bf[...] = v_ref[...].reshape(B * skv, hq * dh).astype(bf)
        xf = x_ref[...].reshape(B * sq, dm).astype(bf)
        nb = B * n_blk

        def chunk_out(wq_c, wo_c, head0):
            q_c = jnp.dot(xf, wq_c, preferred_element_type=jnp.float32)
            k_c = k_bf[:, pl.ds(head0 * dh, dq)]
            v_c = v_bf[:, pl.ds(head0 * dh, dq)]
            ctx_parts = []
            for hh in range(h_per):
                qh = (q_c[:, hh * dh:(hh + 1) * dh].astype(bf)
                      .reshape(nb, BLK, dh))
                kh = k_c[:, hh * dh:(hh + 1) * dh].reshape(nb, BLK, dh)
                vh = v_c[:, hh * dh:(hh + 1) * dh].reshape(nb, BLK, dh)
                s = lax.dot_general(qh, kh, (((2,), (2,)), ((0,), (0,))),
                                    preferred_element_type=jnp.float32)
                w = jnp.exp(s * 0.125)
                w = (w / jnp.sum(w, axis=-1, keepdims=True)).astype(bf)
                ctxh = lax.dot_general(w, vh, (((2,), (1,)), ((0,), (0,))),
                                       preferred_element_type=jnp.float32)
                ctx_parts.append(ctxh.reshape(B * sq, dh))
            ctx = jnp.concatenate(ctx_parts, axis=1).astype(bf)
            return jnp.dot(ctx, wo_c, preferred_element_type=jnp.float32)

        acc = chunk_out(wq_ref[...], wo_ref[...], my * h_per)

        for p in (1, 3, 2):
            org = (my - p) % N_DEV
            for idx, dst in enumerate((
                wq_full.at[:, pl.ds(org * dq, dq)],
                wo_full.at[pl.ds(org * dq, dq), :],
            )):
                recv = pltpu.make_async_remote_copy(
                    src_ref=(wq_ref, wo_ref)[idx], dst_ref=dst,
                    send_sem=send_sems.at[2 * (p - 1) + idx],
                    recv_sem=recv_sems.at[2 * (p - 1) + idx],
                    device_id=((my + p) % N_DEV,),
                    device_id_type=pl.DeviceIdType.MESH,
                )
                recv.wait_recv()
            acc = acc + chunk_out(
                wq_full[:, pl.ds(org * dq, dq)],
                wo_full[pl.ds(org * dq, dq), :],
                org * h_per,
            )

        for rdma in sends:
            rdma.wait_send()
        out_ref[...] = acc.reshape(B, sq, do)

    return pl.pallas_call(
        body,
        out_shape=jax.ShapeDtypeStruct((B, sq, do), jnp.float32),
        in_specs=[pl.BlockSpec(memory_space=pltpu.VMEM)] * 5,
        out_specs=pl.BlockSpec(memory_space=pltpu.VMEM),
        scratch_shapes=[
            pltpu.VMEM((dm, N_DEV * dq), bf),
            pltpu.VMEM((N_DEV * dq, do), bf),
            pltpu.VMEM((B * skv, hq * dh), bf),
            pltpu.VMEM((B * skv, hq * dh), bf),
            pltpu.SemaphoreType.DMA((2 * (N_DEV - 1),)),
            pltpu.SemaphoreType.DMA((2 * (N_DEV - 1),)),
        ],
        compiler_params=pltpu.CompilerParams(collective_id=0),
    )(x, wq_bf, K_ext, V_ext, wo_bf)


# device time: 27646 ns/iter; 1.1099x vs baseline; 1.1099x over previous
import jax
import jax.numpy as jnp
from jax import lax
from jax.experimental import pallas as pl
from jax.experimental.pallas import tpu as pltpu

N_DEV = 4
BLK = 64


def kernel(x, Wq, K_ext, V_ext, Wo):
    B, sq, dm = x.shape
    dq = Wq.shape[1]
    skv, hq, dh = K_ext.shape[1:]
    do = Wo.shape[1]
    h_per = dq // dh
    bf = jnp.bfloat16

    k_t = jnp.transpose(K_ext, (0, 2, 3, 1))
    v_t = jnp.transpose(V_ext, (0, 2, 3, 1))

    def body(x_ref, wq_ref, kt_ref, vt_ref, wo_ref, out_ref,
             wq_full, wo_full, wq_my, wo_my, send_sems, recv_sems):
        my = lax.axis_index("i")

        wq_my[...] = wq_ref[...].astype(bf)
        wo_my[...] = wo_ref[...].astype(bf)

        barrier = pltpu.get_barrier_semaphore()
        for p in range(1, N_DEV):
            pl.semaphore_signal(
                barrier, inc=1,
                device_id=((my + p) % N_DEV,),
                device_id_type=pl.DeviceIdType.MESH,
            )
        pl.semaphore_wait(barrier, N_DEV - 1)

        sends = []
        for p in range(1, N_DEV):
            tgt = (my + p) % N_DEV
            for idx, (src, dst) in enumerate((
                (wq_my, wq_full.at[:, pl.ds(my * dq, dq)]),
                (wo_my, wo_full.at[pl.ds(my * dq, dq), :]),
            )):
                rdma = pltpu.make_async_remote_copy(
                    src_ref=src, dst_ref=dst,
                    send_sem=send_sems.at[2 * (p - 1) + idx],
                    recv_sem=recv_sems.at[2 * (p - 1) + idx],
                    device_id=(tgt,), device_id_type=pl.DeviceIdType.MESH,
                )
                rdma.start()
                sends.append(rdma)

        xf = x_ref[...].reshape(B * sq, dm).astype(bf)
        rows = lax.broadcasted_iota(jnp.int32, (sq, skv), 0) // BLK
        cols = lax.broadcasted_iota(jnp.int32, (sq, skv), 1) // BLK
        mask = rows == cols

        def chunk_out(wq_c, wo_c, head0):
            q_c = jnp.dot(xf, wq_c, preferred_element_type=jnp.float32)
            ctx_parts = []
            for hh in range(h_per):
                per_b = []
                for b in range(B):
                    qhb = (q_c[b * sq:(b + 1) * sq, hh * dh:(hh + 1) * dh]
                           .astype(bf))
                    kthb = kt_ref[b, head0 + hh].astype(bf)
                    vthb = vt_ref[b, head0 + hh].astype(bf)
                    s = jnp.dot(qhb, kthb,
                                preferred_element_type=jnp.float32)
                    w = jnp.where(mask, jnp.exp(s * 0.125), 0.0)
                    w = (w / jnp.sum(w, axis=-1, keepdims=True)).astype(bf)
                    per_b.append(lax.dot_general(
                        w, vthb, (((1,), (1,)), ((), ())),
                        preferred_element_type=jnp.float32))
                ctx_parts.append(jnp.concatenate(per_b, axis=0))
            ctx = jnp.concatenate(ctx_parts, axis=1).astype(bf)
            return jnp.dot(ctx, wo_c, preferred_element_type=jnp.float32)

        acc = chunk_out(wq_my[...], wo_my[...], my * h_per)

        for p in (1, 3, 2):
            org = (my - p) % N_DEV
            for idx, dst in enumerate((
                wq_full.at[:, pl.ds(org * dq, dq)],
                wo_full.at[pl.ds(org * dq, dq), :],
            )):
                recv = pltpu.make_async_remote_copy(
                    src_ref=(wq_my, wo_my)[idx], dst_ref=dst,
                    send_sem=send_sems.at[2 * (p - 1) + idx],
                    recv_sem=recv_sems.at[2 * (p - 1) + idx],
                    device_id=((my + p) % N_DEV,),
                    device_id_type=pl.DeviceIdType.MESH,
                )
                recv.wait_recv()
            acc = acc + chunk_out(
                wq_full[:, pl.ds(org * dq, dq)],
                wo_full[pl.ds(org * dq, dq), :],
                org * h_per,
            )

        for rdma in sends:
            rdma.wait_send()
        out_ref[...] = acc.reshape(B, sq, do)

    return pl.pallas_call(
        body,
        out_shape=jax.ShapeDtypeStruct((B, sq, do), jnp.float32),
        in_specs=[pl.BlockSpec(memory_space=pltpu.VMEM)] * 5,
        out_specs=pl.BlockSpec(memory_space=pltpu.VMEM),
        scratch_shapes=[
            pltpu.VMEM((dm, N_DEV * dq), bf),
            pltpu.VMEM((N_DEV * dq, do), bf),
            pltpu.VMEM((dm, dq), bf),
            pltpu.VMEM((dq, do), bf),
            pltpu.SemaphoreType.DMA((2 * (N_DEV - 1),)),
            pltpu.SemaphoreType.DMA((2 * (N_DEV - 1),)),
        ],
        compiler_params=pltpu.CompilerParams(collective_id=0),
    )(x, Wq, k_t, v_t, Wo)


# device time: 23911 ns/iter; 1.2832x vs baseline; 1.1562x over previous
import jax
import jax.numpy as jnp
from jax import lax
from jax.experimental import pallas as pl
from jax.experimental.pallas import tpu as pltpu

N_DEV = 4
BLK = 64


def kernel(x, Wq, K_ext, V_ext, Wo):
    B, sq, dm = x.shape
    dq = Wq.shape[1]
    skv, hq, dh = K_ext.shape[1:]
    do = Wo.shape[1]
    h_per = dq // dh
    bf = jnp.bfloat16

    k_t = jnp.transpose(K_ext, (0, 2, 3, 1)).astype(bf)
    v_t = jnp.transpose(V_ext, (0, 2, 3, 1)).astype(bf)

    def body(x_ref, wq_ref, kt_ref, vt_ref, wo_ref, out_ref,
             wq_full, wo_full, wq_my, wo_my, send_sems, recv_sems):
        my = lax.axis_index("i")

        wq_my[...] = wq_ref[...].astype(bf)
        wo_my[...] = wo_ref[...].astype(bf)

        barrier = pltpu.get_barrier_semaphore()
        for p in range(1, N_DEV):
            pl.semaphore_signal(
                barrier, inc=1,
                device_id=((my + p) % N_DEV,),
                device_id_type=pl.DeviceIdType.MESH,
            )
        pl.semaphore_wait(barrier, N_DEV - 1)

        sends = []
        for p in range(1, N_DEV):
            tgt = (my + p) % N_DEV
            for idx, (src, dst) in enumerate((
                (wq_my, wq_full.at[:, pl.ds(my * dq, dq)]),
                (wo_my, wo_full.at[pl.ds(my * dq, dq), :]),
            )):
                rdma = pltpu.make_async_remote_copy(
                    src_ref=src, dst_ref=dst,
                    send_sem=send_sems.at[2 * (p - 1) + idx],
                    recv_sem=recv_sems.at[2 * (p - 1) + idx],
                    device_id=(tgt,), device_id_type=pl.DeviceIdType.MESH,
                )
                rdma.start()
                sends.append(rdma)

        xf = x_ref[...].reshape(B * sq, dm).astype(bf)
        rows = lax.broadcasted_iota(jnp.int32, (sq, skv), 0) // BLK
        cols = lax.broadcasted_iota(jnp.int32, (sq, skv), 1) // BLK
        mask = rows == cols

        def chunk_out(wq_c, wo_c, head0):
            q_c = jnp.dot(xf, wq_c, preferred_element_type=jnp.float32)
            ctx_parts = []
            for hh in range(h_per):
                per_b = []
                for b in range(B):
                    qhb = (q_c[b * sq:(b + 1) * sq, hh * dh:(hh + 1) * dh]
                           .astype(bf))
                    kthb = kt_ref[b, head0 + hh]
                    vthb = vt_ref[b, head0 + hh]
                    s = jnp.dot(qhb, kthb,
                                preferred_element_type=jnp.float32)
                    w = jnp.where(mask, jnp.exp(s * 0.125), 0.0)
                    w = (w / jnp.sum(w, axis=-1, keepdims=True)).astype(bf)
                    per_b.append(lax.dot_general(
                        w, vthb, (((1,), (1,)), ((), ())),
                        preferred_element_type=jnp.float32))
                ctx_parts.append(jnp.concatenate(per_b, axis=0))
            ctx = jnp.concatenate(ctx_parts, axis=1).astype(bf)
            return jnp.dot(ctx, wo_c, preferred_element_type=jnp.float32)

        acc = chunk_out(wq_my[...], wo_my[...], my * h_per)

        for p in (1, 3, 2):
            org = (my - p) % N_DEV
            for idx, dst in enumerate((
                wq_full.at[:, pl.ds(org * dq, dq)],
                wo_full.at[pl.ds(org * dq, dq), :],
            )):
                recv = pltpu.make_async_remote_copy(
                    src_ref=(wq_my, wo_my)[idx], dst_ref=dst,
                    send_sem=send_sems.at[2 * (p - 1) + idx],
                    recv_sem=recv_sems.at[2 * (p - 1) + idx],
                    device_id=((my + p) % N_DEV,),
                    device_id_type=pl.DeviceIdType.MESH,
                )
                recv.wait_recv()
            acc = acc + chunk_out(
                wq_full[:, pl.ds(org * dq, dq)],
                wo_full[pl.ds(org * dq, dq), :],
                org * h_per,
            )

        for rdma in sends:
            rdma.wait_send()
        out_ref[...] = acc.reshape(B, sq, do)

    return pl.pallas_call(
        body,
        out_shape=jax.ShapeDtypeStruct((B, sq, do), jnp.float32),
        in_specs=[pl.BlockSpec(memory_space=pltpu.VMEM)] * 5,
        out_specs=pl.BlockSpec(memory_space=pltpu.VMEM),
        scratch_shapes=[
            pltpu.VMEM((dm, N_DEV * dq), bf),
            pltpu.VMEM((N_DEV * dq, do), bf),
            pltpu.VMEM((dm, dq), bf),
            pltpu.VMEM((dq, do), bf),
            pltpu.SemaphoreType.DMA((2 * (N_DEV - 1),)),
            pltpu.SemaphoreType.DMA((2 * (N_DEV - 1),)),
        ],
        compiler_params=pltpu.CompilerParams(collective_id=0),
    )(x, Wq, k_t, v_t, Wo)


# device time: 23902 ns/iter; 1.2837x vs baseline; 1.0004x over previous
import jax
import jax.numpy as jnp
from jax import lax
from jax.experimental import pallas as pl
from jax.experimental.pallas import tpu as pltpu

N_DEV = 4
BLK = 64


def kernel(x, Wq, K_ext, V_ext, Wo):
    B, sq, dm = x.shape
    dq = Wq.shape[1]
    skv, hq, dh = K_ext.shape[1:]
    do = Wo.shape[1]
    h_per = dq // dh
    bf = jnp.bfloat16

    k_t = jnp.transpose(K_ext, (0, 2, 3, 1)).astype(bf)
    v_t = jnp.transpose(V_ext, (0, 2, 3, 1)).astype(bf)

    def body(x_ref, wq_ref, kt_ref, vt_ref, wo_ref, out_ref,
             wq_full, wo_full, wq_my, wo_my, send_sems, recv_sems):
        my = lax.axis_index("i")

        wq_my[...] = wq_ref[...].astype(bf)
        wo_my[...] = wo_ref[...].astype(bf)

        barrier = pltpu.get_barrier_semaphore()
        for p in range(1, N_DEV):
            pl.semaphore_signal(
                barrier, inc=1,
                device_id=((my + p) % N_DEV,),
                device_id_type=pl.DeviceIdType.MESH,
            )
        pl.semaphore_wait(barrier, N_DEV - 1)

        sends = []
        for p in range(1, N_DEV):
            tgt = (my + p) % N_DEV
            for idx, (src, dst) in enumerate((
                (wq_my, wq_full.at[:, pl.ds(my * dq, dq)]),
                (wo_my, wo_full.at[pl.ds(my * dq, dq), :]),
            )):
                rdma = pltpu.make_async_remote_copy(
                    src_ref=src, dst_ref=dst,
                    send_sem=send_sems.at[2 * (p - 1) + idx],
                    recv_sem=recv_sems.at[2 * (p - 1) + idx],
                    device_id=(tgt,), device_id_type=pl.DeviceIdType.MESH,
                )
                rdma.start()
                sends.append(rdma)

        xf = x_ref[...].reshape(B * sq, dm).astype(bf)
        rows = lax.broadcasted_iota(jnp.int32, (sq, skv), 0) // BLK
        cols = lax.broadcasted_iota(jnp.int32, (sq, skv), 1) // BLK
        mask = rows == cols

        def chunk_out(wq_c, wo_c, head0):
            q_c = jnp.dot(xf, wq_c, preferred_element_type=jnp.float32)
            ctx_parts = []
            for hh in range(h_per):
                per_b = []
                for b in range(B):
                    qhb = (q_c[b * sq:(b + 1) * sq, hh * dh:(hh + 1) * dh]
                           .astype(bf))
                    kthb = kt_ref[b, head0 + hh]
                    vthb = vt_ref[b, head0 + hh]
                    s = jnp.dot(qhb, kthb,
                                preferred_element_type=jnp.float32)
                    w = jnp.where(mask, jnp.exp(s * 0.125), 0.0)
                    d = jnp.sum(w, axis=-1, keepdims=True)
                    u = lax.dot_general(
                        w.astype(bf), vthb, (((1,), (1,)), ((), ())),
                        preferred_element_type=jnp.float32)
                    per_b.append(u / d)
                ctx_parts.append(jnp.concatenate(per_b, axis=0))
            ctx = jnp.concatenate(ctx_parts, axis=1).astype(bf)
            return jnp.dot(ctx, wo_c, preferred_element_type=jnp.float32)

        acc = chunk_out(wq_my[...], wo_my[...], my * h_per)

        for p in (1, 3, 2):
            org = (my - p) % N_DEV
            for idx, dst in enumerate((
                wq_full.at[:, pl.ds(org * dq, dq)],
                wo_full.at[pl.ds(org * dq, dq), :],
            )):
                recv = pltpu.make_async_remote_copy(
                    src_ref=(wq_my, wo_my)[idx], dst_ref=dst,
                    send_sem=send_sems.at[2 * (p - 1) + idx],
                    recv_sem=recv_sems.at[2 * (p - 1) + idx],
                    device_id=((my + p) % N_DEV,),
                    device_id_type=pl.DeviceIdType.MESH,
                )
                recv.wait_recv()
            acc = acc + chunk_out(
                wq_full[:, pl.ds(org * dq, dq)],
                wo_full[pl.ds(org * dq, dq), :],
                org * h_per,
            )

        for rdma in sends:
            rdma.wait_send()
        out_ref[...] = acc.reshape(B, sq, do)

    return pl.pallas_call(
        body,
        out_shape=jax.ShapeDtypeStruct((B, sq, do), jnp.float32),
        in_specs=[pl.BlockSpec(memory_space=pltpu.VMEM)] * 5,
        out_specs=pl.BlockSpec(memory_space=pltpu.VMEM),
        scratch_shapes=[
            pltpu.VMEM((dm, N_DEV * dq), bf),
            pltpu.VMEM((N_DEV * dq, do), bf),
            pltpu.VMEM((dm, dq), bf),
            pltpu.VMEM((dq, do), bf),
            pltpu.SemaphoreType.DMA((2 * (N_DEV - 1),)),
            pltpu.SemaphoreType.DMA((2 * (N_DEV - 1),)),
        ],
        compiler_params=pltpu.CompilerParams(collective_id=0),
    )(x, Wq, k_t, v_t, Wo)
